# baseline (speedup 1.0000x reference)
import functools

import jax
import jax.numpy as jnp
import numpy as np
from jax.experimental import pallas as pl
from jax.experimental.pallas import tpu as pltpu
from jax.sharding import Mesh, PartitionSpec as P

try:
    from jax import shard_map as _shard_map
except ImportError:
    from jax.experimental.shard_map import shard_map as _shard_map

_BM, _BN, _BK = 2048, 2048, 512
_INV_SQRT2 = 0.7071067811865476


def _matmul_gelu_kernel(a_ref, w_ref, b_ref, o_ref, *, k_steps):
    k = pl.program_id(2)

    @pl.when(k == 0)
    def _first():
        o_ref[...] = jnp.dot(a_ref[...], w_ref[...],
                             preferred_element_type=jnp.float32) + b_ref[...]

    @pl.when(k > 0)
    def _rest():
        o_ref[...] += jnp.dot(a_ref[...], w_ref[...],
                              preferred_element_type=jnp.float32)

    @pl.when(k == k_steps - 1)
    def _finish():
        x = o_ref[...]
        o_ref[...] = x * (0.5 * (1.0 + jax.lax.erf(x * _INV_SQRT2)))


def _fused_matmul_gelu(hidden_states, W, b):
    batch, seq, d_in = hidden_states.shape
    m = batch * seq
    k_dim, n = W.shape
    a = hidden_states.reshape(m, d_in)
    b2 = b.reshape(1, n)

    bm, bn, bk = min(_BM, m), min(_BN, n), min(_BK, k_dim)
    k_steps = k_dim // bk
    grid = (m // bm, n // bn, k_steps)

    out = pl.pallas_call(
        functools.partial(_matmul_gelu_kernel, k_steps=k_steps),
        grid=grid,
        in_specs=[
            pl.BlockSpec((bm, bk), lambda mi, ni, ki: (mi, ki)),
            pl.BlockSpec((bk, bn), lambda mi, ni, ki: (ki, ni)),
            pl.BlockSpec((1, bn), lambda mi, ni, ki: (0, ni)),
        ],
        out_specs=pl.BlockSpec((bm, bn), lambda mi, ni, ki: (mi, ni)),
        out_shape=jax.ShapeDtypeStruct((m, n), jnp.float32),
        compiler_params=pltpu.CompilerParams(
            dimension_semantics=("parallel", "parallel", "arbitrary"),
        ),
    )(a, W, b2)
    return out.reshape(batch, seq, n)


def kernel(hidden_states, W, b):
    devs = jax.devices()
    n_cols = W.shape[1]
    if len(devs) < 2 or n_cols % (2 * _BN) != 0:
        return _fused_matmul_gelu(hidden_states, W, b)
    mesh = Mesh(np.array(devs[:2]), ("x",))
    sharded = _shard_map(
        _fused_matmul_gelu,
        mesh=mesh,
        in_specs=(P(), P(None, "x"), P("x")),
        out_specs=P(None, None, "x"),
        check_vma=False,
    )
    return sharded(hidden_states, W, b)


# speedup vs baseline: 1.3537x; 1.3537x over previous
import functools

import jax
import jax.numpy as jnp
from jax.experimental import pallas as pl
from jax.experimental.pallas import tpu as pltpu

_BM, _BN, _BK = 2048, 2048, 512
_INV_SQRT2 = 0.7071067811865476


def _matmul_gelu_kernel(a_ref, w_ref, b_ref, o_ref, *, k_steps):
    if k_steps == 1:
        x = jnp.dot(a_ref[...], w_ref[...],
                    preferred_element_type=jnp.float32) + b_ref[...]
        o_ref[...] = x * (0.5 * (1.0 + jax.lax.erf(x * _INV_SQRT2)))
        return

    k = pl.program_id(2)

    @pl.when(k == 0)
    def _first():
        o_ref[...] = jnp.dot(a_ref[...], w_ref[...],
                             preferred_element_type=jnp.float32) + b_ref[...]

    @pl.when(jnp.logical_and(k > 0, k < k_steps - 1))
    def _middle():
        o_ref[...] += jnp.dot(a_ref[...], w_ref[...],
                              preferred_element_type=jnp.float32)

    @pl.when(k == k_steps - 1)
    def _finish():
        x = o_ref[...] + jnp.dot(a_ref[...], w_ref[...],
                                 preferred_element_type=jnp.float32)
        o_ref[...] = x * (0.5 * (1.0 + jax.lax.erf(x * _INV_SQRT2)))


def kernel(hidden_states, W, b):
    batch, seq, d_in = hidden_states.shape
    m = batch * seq
    k_dim, n = W.shape
    a = hidden_states.reshape(m, d_in)
    b2 = b.reshape(1, n)

    bm, bn, bk = min(_BM, m), min(_BN, n), min(_BK, k_dim)
    k_steps = k_dim // bk
    grid = (m // bm, n // bn, k_steps)

    out = pl.pallas_call(
        functools.partial(_matmul_gelu_kernel, k_steps=k_steps),
        grid=grid,
        in_specs=[
            pl.BlockSpec((bm, bk), lambda mi, ni, ki: (mi, ki)),
            pl.BlockSpec((bk, bn), lambda mi, ni, ki: (ki, ni)),
            pl.BlockSpec((1, bn), lambda mi, ni, ki: (0, ni)),
        ],
        out_specs=pl.BlockSpec((bm, bn), lambda mi, ni, ki: (mi, ni)),
        out_shape=jax.ShapeDtypeStruct((m, n), jnp.float32),
        compiler_params=pltpu.CompilerParams(
            dimension_semantics=("parallel", "parallel", "arbitrary"),
        ),
    )(a, W, b2)
    return out.reshape(batch, seq, n)
